# baseline (speedup 1.0000x reference)
import functools

import jax
import jax.numpy as jnp
from jax.experimental import pallas as pl
from jax.experimental.pallas import tpu as pltpu

_BM = 1024
_BK = 2048


def _nnue_body(wf_ref, bf_ref, w0w_ref, w0b_ref, turn_ref,
               l0b2_ref, l1w_ref, l1b_ref, l2w_ref, l2b_ref,
               out_ref, accw_ref, accb_ref, *, nk):
    k = pl.program_id(1)

    @pl.when(k == 0)
    def _init():
        accw_ref[...] = jnp.zeros_like(accw_ref)
        accb_ref[...] = jnp.zeros_like(accb_ref)

    dims = (((1,), (1,)), ((), ()))
    accw_ref[...] += jax.lax.dot_general(
        wf_ref[...], w0w_ref[...], dims, precision=jax.lax.Precision.HIGHEST, preferred_element_type=jnp.float32)
    accb_ref[...] += jax.lax.dot_general(
        bf_ref[...], w0b_ref[...], dims, precision=jax.lax.Precision.HIGHEST, preferred_element_type=jnp.float32)

    @pl.when(k == nk - 1)
    def _tail():
        wb = accw_ref[...] + accb_ref[...]
        bw = jnp.concatenate([wb[:, 4:], wb[:, :4]], axis=1)
        t = turn_ref[...]
        acc = t * wb + (1.0 - t) * bw + l0b2_ref[...]
        l1_x = jnp.clip(acc, 0.0, 1.0)
        h = jax.lax.dot_general(l1_x, l1w_ref[...], (((1,), (0,)), ((), ())),
                                preferred_element_type=jnp.float32)
        h = jnp.clip(h + l1b_ref[...], 0.0, 1.0)
        out_ref[...] = jnp.sum(h * l2w_ref[...], axis=1, keepdims=True) \
            + l2b_ref[...]


@jax.jit
def kernel(white_features, black_features, turn, score, result,
           l0_w, l0_b, l1_w, l1_b, l2_w, l2_b):
    B, K = white_features.shape
    M = l0_w.shape[0]

    bm, bk = _BM, _BK
    nm, nk = B // bm, K // bk

    zeros = jnp.zeros_like(l0_w)
    w0w = jnp.concatenate([l0_w, zeros], axis=0)
    w0b = jnp.concatenate([zeros, l0_w], axis=0)
    l0b2 = jnp.concatenate([l0_b, l0_b]).reshape(1, 2 * M)
    l1b2 = l1_b.reshape(1, -1)
    l2w2 = l2_w.reshape(1, -1)
    l2b2 = l2_b.reshape(1, 1)

    out = pl.pallas_call(
        functools.partial(_nnue_body, nk=nk),
        grid=(nm, nk),
        in_specs=[
            pl.BlockSpec((bm, bk), lambda m, k: (m, k)),
            pl.BlockSpec((bm, bk), lambda m, k: (m, k)),
            pl.BlockSpec((2 * M, bk), lambda m, k: (0, k)),
            pl.BlockSpec((2 * M, bk), lambda m, k: (0, k)),
            pl.BlockSpec((bm, 1), lambda m, k: (m, 0)),
            pl.BlockSpec((1, 2 * M), lambda m, k: (0, 0)),
            pl.BlockSpec(l1_w.T.shape, lambda m, k: (0, 0)),
            pl.BlockSpec((1, 2 * M), lambda m, k: (0, 0)),
            pl.BlockSpec((1, 2 * M), lambda m, k: (0, 0)),
            pl.BlockSpec((1, 1), lambda m, k: (0, 0)),
        ],
        out_specs=pl.BlockSpec((bm, 1), lambda m, k: (m, 0)),
        out_shape=jax.ShapeDtypeStruct((B, 1), jnp.float32),
        scratch_shapes=[
            pltpu.VMEM((bm, 2 * M), jnp.float32),
            pltpu.VMEM((bm, 2 * M), jnp.float32),
        ],
        compiler_params=pltpu.CompilerParams(
            dimension_semantics=("parallel", "arbitrary"),
        ),
    )(white_features, black_features, w0w, w0b, turn,
      l0b2, l1_w.T, l1b2, l2w2, l2b2)
    return out


# speedup vs baseline: 2.5590x; 2.5590x over previous
import functools

import jax
import jax.numpy as jnp
from jax.experimental import pallas as pl
from jax.experimental.pallas import tpu as pltpu

_BM = 1024
_BK = 2048


def _nnue_body(wf_ref, bf_ref, w0w_ref, w0b_ref, turn_ref,
               l0b2_ref, l1w_ref, l1b_ref, l2w_ref, l2b_ref,
               out_ref, accw_ref, accb_ref, *, nk):
    k = pl.program_id(1)

    @pl.when(k == 0)
    def _init():
        accw_ref[...] = jnp.zeros_like(accw_ref)
        accb_ref[...] = jnp.zeros_like(accb_ref)

    dims = (((1,), (1,)), ((), ()))
    accw_ref[...] += jax.lax.dot_general(
        wf_ref[...], w0w_ref[...], dims, preferred_element_type=jnp.float32)
    accb_ref[...] += jax.lax.dot_general(
        bf_ref[...], w0b_ref[...], dims, preferred_element_type=jnp.float32)

    @pl.when(k == nk - 1)
    def _tail():
        wb = accw_ref[...] + accb_ref[...]
        bw = jnp.concatenate([wb[:, 4:], wb[:, :4]], axis=1)
        t = turn_ref[...]
        acc = t * wb + (1.0 - t) * bw + l0b2_ref[...]
        l1_x = jnp.clip(acc, 0.0, 1.0)
        h = jax.lax.dot_general(l1_x, l1w_ref[...], (((1,), (0,)), ((), ())),
                                preferred_element_type=jnp.float32)
        h = jnp.clip(h + l1b_ref[...], 0.0, 1.0)
        out_ref[...] = jnp.sum(h * l2w_ref[...], axis=1, keepdims=True) \
            + l2b_ref[...]


@jax.jit
def kernel(white_features, black_features, turn, score, result,
           l0_w, l0_b, l1_w, l1_b, l2_w, l2_b):
    B, K = white_features.shape
    M = l0_w.shape[0]

    bm, bk = _BM, _BK
    nm, nk = B // bm, K // bk

    zeros = jnp.zeros_like(l0_w)
    w0w = jnp.concatenate([l0_w, zeros], axis=0)
    w0b = jnp.concatenate([zeros, l0_w], axis=0)
    l0b2 = jnp.concatenate([l0_b, l0_b]).reshape(1, 2 * M)
    l1b2 = l1_b.reshape(1, -1)
    l2w2 = l2_w.reshape(1, -1)
    l2b2 = l2_b.reshape(1, 1)

    out = pl.pallas_call(
        functools.partial(_nnue_body, nk=nk),
        grid=(nm, nk),
        in_specs=[
            pl.BlockSpec((bm, bk), lambda m, k: (m, k)),
            pl.BlockSpec((bm, bk), lambda m, k: (m, k)),
            pl.BlockSpec((2 * M, bk), lambda m, k: (0, k)),
            pl.BlockSpec((2 * M, bk), lambda m, k: (0, k)),
            pl.BlockSpec((bm, 1), lambda m, k: (m, 0)),
            pl.BlockSpec((1, 2 * M), lambda m, k: (0, 0)),
            pl.BlockSpec(l1_w.T.shape, lambda m, k: (0, 0)),
            pl.BlockSpec((1, 2 * M), lambda m, k: (0, 0)),
            pl.BlockSpec((1, 2 * M), lambda m, k: (0, 0)),
            pl.BlockSpec((1, 1), lambda m, k: (0, 0)),
        ],
        out_specs=pl.BlockSpec((bm, 1), lambda m, k: (m, 0)),
        out_shape=jax.ShapeDtypeStruct((B, 1), jnp.float32),
        scratch_shapes=[
            pltpu.VMEM((bm, 2 * M), jnp.float32),
            pltpu.VMEM((bm, 2 * M), jnp.float32),
        ],
        compiler_params=pltpu.CompilerParams(
            dimension_semantics=("parallel", "arbitrary"),
        ),
    )(white_features, black_features, w0w, w0b, turn,
      l0b2, l1_w.T, l1b2, l2w2, l2b2)
    return out
